# baseline (speedup 1.0000x reference)
import functools

import jax
import jax.numpy as jnp
from jax import lax
from jax.experimental import pallas as pl
from jax.experimental.pallas import tpu as pltpu
from jax.experimental.pallas import tpu_sc as plsc

N_ROWS = 100000
BM = 2000
EMB = 128
FEAT = 1024

_NC, _NS = 2, 16
_NW = _NC * _NS
_CHUNK = 1000
_NCHUNKS = N_ROWS // _CHUNK


def _tc_body(raw_ref, i_ref, w_ref, b_ref, io_ref):
    io_ref[:, :EMB] = i_ref[...]
    proj = jnp.dot(raw_ref[...], w_ref[...], preferred_element_type=jnp.float32)
    io_ref[:, EMB:] = proj + b_ref[...]


def _sc_copy_body(u_hbm, out_hbm):
    wid = lax.axis_index("s") * _NC + lax.axis_index("c")
    n_full, rem = divmod(_NCHUNKS, _NW)
    n_mine = jnp.where(wid < rem, n_full + 1, n_full)

    def body(t, carry):
        base = (wid + _NW * t) * _CHUNK
        pltpu.sync_copy(u_hbm.at[pl.ds(base, _CHUNK)],
                        out_hbm.at[pl.ds(base, _CHUNK)])
        return carry

    lax.fori_loop(0, n_mine, body, 0)


_sc_copy = functools.partial(
    pl.kernel,
    mesh=plsc.VectorSubcoreMesh(core_axis_name="c", subcore_axis_name="s"),
    out_type=jax.ShapeDtypeStruct((N_ROWS, 2 * EMB), jnp.float32),
)(_sc_copy_body)


def kernel(item_raw_features, u_embedding, i_embedding, W, b):
    b2 = b.reshape(1, EMB)
    item_e = pl.pallas_call(
        _tc_body,
        grid=(N_ROWS // BM,),
        in_specs=[
            pl.BlockSpec((BM, FEAT), lambda i: (i, 0)),
            pl.BlockSpec((BM, EMB), lambda i: (i, 0)),
            pl.BlockSpec((FEAT, EMB), lambda i: (0, 0)),
            pl.BlockSpec((1, EMB), lambda i: (0, 0)),
        ],
        out_specs=pl.BlockSpec((BM, 2 * EMB), lambda i: (i, 0)),
        out_shape=jax.ShapeDtypeStruct((N_ROWS, 2 * EMB), jnp.float32),
        compiler_params=pltpu.CompilerParams(
            dimension_semantics=("arbitrary",),
        ),
    )(item_raw_features, i_embedding, W, b2)
    user_e = _sc_copy(u_embedding)
    return (user_e, item_e)


# speedup vs baseline: 12.2594x; 12.2594x over previous
import functools

import jax
import jax.numpy as jnp
from jax import lax
from jax.experimental import pallas as pl
from jax.experimental.pallas import tpu as pltpu
from jax.experimental.pallas import tpu_sc as plsc

N_ROWS = 100000
BM = 2000
EMB = 128
FEAT = 1024

_NC, _NS = 2, 16
_NW = _NC * _NS
_CHUNK = 200
_NCHUNKS = N_ROWS // _CHUNK


def _tc_body(raw_ref, i_ref, w_ref, b_ref, io_ref):
    io_ref[:, :EMB] = i_ref[...]
    proj = jnp.dot(raw_ref[...], w_ref[...], preferred_element_type=jnp.float32)
    io_ref[:, EMB:] = proj + b_ref[...]


def _sc_copy_body(u_hbm, out_hbm, buf):
    wid = lax.axis_index("s") * _NC + lax.axis_index("c")
    n_full, rem = divmod(_NCHUNKS, _NW)
    n_mine = jnp.where(wid < rem, n_full + 1, n_full)

    def body(t, carry):
        base = (wid + _NW * t) * _CHUNK
        pltpu.sync_copy(u_hbm.at[pl.ds(base, _CHUNK)], buf)
        pltpu.sync_copy(buf, out_hbm.at[pl.ds(base, _CHUNK)])
        return carry

    lax.fori_loop(0, n_mine, body, 0)


_sc_copy = functools.partial(
    pl.kernel,
    mesh=plsc.VectorSubcoreMesh(core_axis_name="c", subcore_axis_name="s"),
    out_type=jax.ShapeDtypeStruct((N_ROWS, 2 * EMB), jnp.float32),
    scratch_types=[pltpu.VMEM((_CHUNK, 2 * EMB), jnp.float32)],
)(_sc_copy_body)


def kernel(item_raw_features, u_embedding, i_embedding, W, b):
    b2 = b.reshape(1, EMB)
    item_e = pl.pallas_call(
        _tc_body,
        grid=(N_ROWS // BM,),
        in_specs=[
            pl.BlockSpec((BM, FEAT), lambda i: (i, 0)),
            pl.BlockSpec((BM, EMB), lambda i: (i, 0)),
            pl.BlockSpec((FEAT, EMB), lambda i: (0, 0)),
            pl.BlockSpec((1, EMB), lambda i: (0, 0)),
        ],
        out_specs=pl.BlockSpec((BM, 2 * EMB), lambda i: (i, 0)),
        out_shape=jax.ShapeDtypeStruct((N_ROWS, 2 * EMB), jnp.float32),
        compiler_params=pltpu.CompilerParams(
            dimension_semantics=("arbitrary",),
        ),
    )(item_raw_features, i_embedding, W, b2)
    user_e = _sc_copy(u_embedding)
    return (user_e, item_e)


# speedup vs baseline: 13.4432x; 1.0966x over previous
import jax
import jax.numpy as jnp
from jax.experimental import pallas as pl
from jax.experimental.pallas import tpu as pltpu

N_ROWS = 100000
BM = 2000
EMB = 128
FEAT = 1024


def _body(raw_ref, u_ref, i_ref, w_ref, b_ref, uo_ref, io_ref):
    uo_ref[...] = u_ref[...]
    io_ref[:, :EMB] = i_ref[...]
    proj = jnp.dot(raw_ref[...], w_ref[...], preferred_element_type=jnp.float32)
    io_ref[:, EMB:] = proj + b_ref[...]


def kernel(item_raw_features, u_embedding, i_embedding, W, b):
    b2 = b.reshape(1, EMB)
    grid = (N_ROWS // BM,)
    user_e, item_e = pl.pallas_call(
        _body,
        grid=grid,
        in_specs=[
            pl.BlockSpec((BM, FEAT), lambda i: (i, 0)),
            pl.BlockSpec((BM, 2 * EMB), lambda i: (i, 0)),
            pl.BlockSpec((BM, EMB), lambda i: (i, 0)),
            pl.BlockSpec((FEAT, EMB), lambda i: (0, 0)),
            pl.BlockSpec((1, EMB), lambda i: (0, 0)),
        ],
        out_specs=[
            pl.BlockSpec((BM, 2 * EMB), lambda i: (i, 0)),
            pl.BlockSpec((BM, 2 * EMB), lambda i: (i, 0)),
        ],
        out_shape=[
            jax.ShapeDtypeStruct((N_ROWS, 2 * EMB), jnp.float32),
            jax.ShapeDtypeStruct((N_ROWS, 2 * EMB), jnp.float32),
        ],
        compiler_params=pltpu.CompilerParams(
            dimension_semantics=("arbitrary",),
        ),
    )(item_raw_features, u_embedding, i_embedding, W, b2)
    return (user_e, item_e)
